# Initial kernel scaffold; baseline (speedup 1.0000x reference)
import functools

import jax
import jax.numpy as jnp
from jax import lax
from jax.experimental import pallas as pl
from jax.experimental.pallas import tpu as pltpu
from jax.experimental.pallas import tpu_sc as plsc

B, F, L, D = 1024, 26, 50, 32
LP = 56
G = B * F
NW = 32
GPW = G // NW
CG = 2
IPC = CG * LP
OPC = CG * D * L
NCHUNK = GPW // CG

_mesh = plsc.VectorSubcoreMesh(core_axis_name="c", subcore_axis_name="s")


@functools.partial(
    pl.kernel,
    out_type=jax.ShapeDtypeStruct((G * D * L,), jnp.float32),
    mesh=_mesh,
    scratch_types=[
        pltpu.VMEM((NCHUNK, IPC), jnp.int32),
        pltpu.VMEM((IPC, D), jnp.float32),
        pltpu.VMEM((OPC,), jnp.float32),
        pltpu.SemaphoreType.DMA,
    ],
)
def _emb_lookup(idx_hbm, table_hbm, out_hbm, idx_v, rows_v, out_v, sem):
    wid = lax.axis_index("s") * 2 + lax.axis_index("c")
    iota = lax.iota(jnp.int32, 16)
    iota50 = iota * 50

    pltpu.sync_copy(idx_hbm.at[pl.ds(wid * NCHUNK, NCHUNK)], idx_v)

    def chunk_body(c, _):
        pltpu.async_copy(table_hbm.at[idx_v.at[c]], rows_v, sem).wait()

        def row_body(l, _):
            for g in range(CG):
                r = jnp.full((16,), g * LP + l, dtype=jnp.int32)
                v0 = plsc.load_gather(rows_v, [r, iota])
                v1 = plsc.load_gather(rows_v, [r, iota + 16])
                o0 = jnp.full((16,), g * (D * L) + l, dtype=jnp.int32) + iota50
                plsc.store_scatter(out_v, [o0], v0)
                plsc.store_scatter(out_v, [o0 + 16 * 50], v1)
            return 0

        lax.fori_loop(0, L, row_body, 0, unroll=2)

        obase = (wid * GPW + c * CG) * (D * L)
        pltpu.sync_copy(out_v, out_hbm.at[pl.ds(obase, OPC)])
        return 0

    lax.fori_loop(0, NCHUNK, chunk_body, 0)


def kernel(inputs, table):
    idx = inputs.reshape(G, L).astype(jnp.int32)
    idx = jnp.pad(idx, ((0, 0), (0, LP - L)))
    flat = _emb_lookup(idx, table)
    return flat.reshape(B, F, D, L)


# speedup vs baseline: 1.7068x; 1.7068x over previous
import functools

import jax
import jax.numpy as jnp
from jax import lax
from jax.experimental import pallas as pl
from jax.experimental.pallas import tpu as pltpu
from jax.experimental.pallas import tpu_sc as plsc

B, F, L, D = 1024, 26, 50, 32
LP = 56
G = B * F
NW = 32
GPW = G // NW
CG = 2
IPC = CG * LP
OPC = CG * D * L
NCHUNK = GPW // CG

_mesh = plsc.VectorSubcoreMesh(core_axis_name="c", subcore_axis_name="s")


@functools.partial(
    pl.kernel,
    out_type=jax.ShapeDtypeStruct((G * D * L,), jnp.float32),
    mesh=_mesh,
    scratch_types=[
        pltpu.VMEM((NCHUNK, IPC), jnp.int32),
        pltpu.VMEM((IPC, D), jnp.float32),
        pltpu.VMEM((OPC,), jnp.float32),
        pltpu.SemaphoreType.DMA,
    ],
    compiler_params=pltpu.CompilerParams(
        needs_layout_passes=False, use_tc_tiling_on_sc=False
    ),
)
def _emb_lookup(idx_hbm, table_hbm, out_hbm, idx_v, rows_v, out_v, sem):
    wid = lax.axis_index("s") * 2 + lax.axis_index("c")
    iota = lax.iota(jnp.int32, 16)
    iota50 = iota * 50

    pltpu.sync_copy(idx_hbm.at[pl.ds(wid * NCHUNK, NCHUNK)], idx_v)

    def chunk_body(c, _):
        pltpu.async_copy(table_hbm.at[idx_v.at[c]], rows_v, sem).wait()

        def row_body(l, _):
            for g in range(CG):
                r = g * LP + l
                v0 = rows_v[r, pl.ds(0, 16)]
                v1 = rows_v[r, pl.ds(16, 16)]
                o0 = jnp.full((16,), g * (D * L) + l, dtype=jnp.int32) + iota50
                plsc.store_scatter(out_v, [o0], v0)
                plsc.store_scatter(out_v, [o0 + 16 * 50], v1)
            return 0

        lax.fori_loop(0, L, row_body, 0, unroll=2)

        obase = (wid * GPW + c * CG) * (D * L)
        pltpu.sync_copy(out_v, out_hbm.at[pl.ds(obase, OPC)])
        return 0

    lax.fori_loop(0, NCHUNK, chunk_body, 0)


def kernel(inputs, table):
    idx = inputs.reshape(G, L).astype(jnp.int32)
    idx = jnp.pad(idx, ((0, 0), (0, LP - L))).reshape(G // CG, IPC)
    flat = _emb_lookup(idx, table)
    return flat.reshape(B, F, D, L)


# speedup vs baseline: 3.6709x; 2.1507x over previous
import functools

import jax
import jax.numpy as jnp
from jax import lax
from jax.experimental import pallas as pl
from jax.experimental.pallas import tpu as pltpu
from jax.experimental.pallas import tpu_sc as plsc

B, F, L, D = 1024, 26, 50, 32
G = B * F
NW = 32
GPW = G // NW
CG = 2
IPC = CG * L
OPC = CG * D * L
NCHUNK = GPW // CG
NBUF = 4

_mesh = plsc.VectorSubcoreMesh(core_axis_name="c", subcore_axis_name="s")


@functools.partial(
    pl.kernel,
    out_type=jax.ShapeDtypeStruct((G * D * L,), jnp.float32),
    mesh=_mesh,
    scratch_types=[
        pltpu.VMEM((NCHUNK, IPC), jnp.int32),
        pltpu.VMEM((NBUF, IPC, D), jnp.float32),
        pltpu.VMEM((NBUF, OPC), jnp.float32),
    ]
    + [pltpu.SemaphoreType.DMA] * (2 * NBUF),
    compiler_params=pltpu.CompilerParams(
        needs_layout_passes=False, use_tc_tiling_on_sc=False
    ),
)
def _emb_lookup(idx_hbm, table_hbm, out_hbm, idx_v, rows_v, out_v, *sems):
    sem_g = sems[:NBUF]
    sem_w = sems[NBUF:]
    wid = lax.axis_index("s") * 2 + lax.axis_index("c")
    iota50 = lax.iota(jnp.int32, 16) * 50

    def out_slice(c):
        return out_hbm.at[pl.ds((wid * GPW + c * CG) * (D * L), OPC)]

    pltpu.sync_copy(idx_hbm.at[pl.ds(wid * NCHUNK, NCHUNK)], idx_v)

    for b in range(NBUF):
        pltpu.async_copy(table_hbm.at[idx_v.at[b]], rows_v.at[b], sem_g[b])

    def quad_body(p, _):
        for b in range(NBUF):
            c = p * NBUF + b
            rv = rows_v.at[b]
            ov = out_v.at[b]

            pltpu.make_async_copy(table_hbm.at[idx_v.at[c]], rv, sem_g[b]).wait()

            @pl.when(p > 0)
            def _():
                pltpu.make_async_copy(ov, out_slice(c - NBUF), sem_w[b]).wait()

            def row_body(l, _):
                for g in range(CG):
                    r = g * L + l
                    v0 = rv[r, pl.ds(0, 16)]
                    v1 = rv[r, pl.ds(16, 16)]
                    o0 = jnp.full((16,), g * (D * L) + l, dtype=jnp.int32) + iota50
                    plsc.store_scatter(ov, [o0], v0)
                    plsc.store_scatter(ov, [o0 + 16 * 50], v1)
                return 0

            lax.fori_loop(0, L, row_body, 0, unroll=2)

            @pl.when(c + NBUF < NCHUNK)
            def _():
                pltpu.async_copy(table_hbm.at[idx_v.at[c + NBUF]], rv, sem_g[b])

            pltpu.async_copy(ov, out_slice(c), sem_w[b])
        return 0

    lax.fori_loop(0, NCHUNK // NBUF, quad_body, 0)

    for b in range(NBUF):
        c = NCHUNK - NBUF + b
        pltpu.make_async_copy(out_v.at[b], out_slice(c), sem_w[b]).wait()


def kernel(inputs, table):
    idx = inputs.reshape(G // CG, IPC).astype(jnp.int32)
    flat = _emb_lookup(idx, table)
    return flat.reshape(B, F, D, L)


# speedup vs baseline: 4.2066x; 1.1460x over previous
import functools

import jax
import jax.numpy as jnp
from jax import lax
from jax.experimental import pallas as pl
from jax.experimental.pallas import tpu as pltpu
from jax.experimental.pallas import tpu_sc as plsc

B, F, L, D = 1024, 26, 50, 32
G = B * F
NW = 32
GPW = G // NW
CG = 2
IPC = CG * L
OPC = CG * D * L
NCHUNK = GPW // CG
NBUF = 4

_mesh = plsc.VectorSubcoreMesh(core_axis_name="c", subcore_axis_name="s")


BPW = B // NW
CPB = F // CG


@functools.partial(
    pl.kernel,
    out_type=jax.ShapeDtypeStruct((B, F, D, L), jnp.float32),
    mesh=_mesh,
    scratch_types=[
        pltpu.VMEM((NCHUNK, IPC), jnp.int32),
        pltpu.VMEM((NBUF, IPC, D), jnp.float32),
        pltpu.VMEM((NBUF, CG, D, L), jnp.float32),
    ]
    + [pltpu.SemaphoreType.DMA] * (2 * NBUF),
    compiler_params=pltpu.CompilerParams(
        needs_layout_passes=False, use_tc_tiling_on_sc=False
    ),
)
def _emb_lookup(idx_hbm, table_hbm, out_hbm, idx_v, rows_v, out_v, *sems):
    sem_g = sems[:NBUF]
    sem_w = sems[NBUF:]
    wid = lax.axis_index("s") * 2 + lax.axis_index("c")
    iota16 = lax.iota(jnp.int32, 16)

    def out_slice(c):
        b0 = wid * BPW + c // CPB
        f0 = (c % CPB) * CG
        return out_hbm.at[b0, pl.ds(f0, CG)]

    pltpu.sync_copy(idx_hbm.at[pl.ds(wid * NCHUNK, NCHUNK)], idx_v)

    for b in range(NBUF):
        pltpu.async_copy(table_hbm.at[idx_v.at[b]], rows_v.at[b], sem_g[b])

    def quad_body(p, _):
        for b in range(NBUF):
            c = p * NBUF + b
            rv = rows_v.at[b]
            ov = out_v.at[b]

            pltpu.make_async_copy(table_hbm.at[idx_v.at[c]], rv, sem_g[b]).wait()

            @pl.when(p > 0)
            def _():
                pltpu.make_async_copy(ov, out_slice(c - NBUF), sem_w[b]).wait()

            def row_body(l, _):
                ol = jnp.full((16,), l, dtype=jnp.int32)
                for g in range(CG):
                    r = g * L + l
                    og = jnp.full((16,), g, dtype=jnp.int32)
                    v0 = rv[r, pl.ds(0, 16)]
                    v1 = rv[r, pl.ds(16, 16)]
                    plsc.store_scatter(ov, [og, iota16, ol], v0)
                    plsc.store_scatter(ov, [og, iota16 + 16, ol], v1)
                return 0

            lax.fori_loop(0, L, row_body, 0, unroll=2)

            @pl.when(c + NBUF < NCHUNK)
            def _():
                pltpu.async_copy(table_hbm.at[idx_v.at[c + NBUF]], rv, sem_g[b])

            pltpu.async_copy(ov, out_slice(c), sem_w[b])
        return 0

    lax.fori_loop(0, NCHUNK // NBUF, quad_body, 0)

    for b in range(NBUF):
        c = NCHUNK - NBUF + b
        pltpu.make_async_copy(out_v.at[b], out_slice(c), sem_w[b]).wait()


def kernel(inputs, table):
    idx = inputs.reshape(G // CG, IPC).astype(jnp.int32)
    return _emb_lookup(idx, table)


# speedup vs baseline: 4.2080x; 1.0003x over previous
import functools

import jax
import jax.numpy as jnp
from jax import lax
from jax.experimental import pallas as pl
from jax.experimental.pallas import tpu as pltpu
from jax.experimental.pallas import tpu_sc as plsc

B, F, L, D = 1024, 26, 50, 32
G = B * F
NW = 32
GPW = G // NW
CG = 2
IPC = CG * L
OPC = CG * D * L
NCHUNK = GPW // CG
NBUF = 8

_mesh = plsc.VectorSubcoreMesh(core_axis_name="c", subcore_axis_name="s")


BPW = B // NW
CPB = F // CG


@functools.partial(
    pl.kernel,
    out_type=jax.ShapeDtypeStruct((B, F, D, L), jnp.float32),
    mesh=_mesh,
    scratch_types=[
        pltpu.VMEM((NCHUNK, IPC), jnp.int32),
        pltpu.VMEM((NBUF, IPC, D), jnp.float32),
        pltpu.VMEM((NBUF, CG, D, L), jnp.float32),
    ]
    + [pltpu.SemaphoreType.DMA] * (2 * NBUF),
    compiler_params=pltpu.CompilerParams(
        needs_layout_passes=False, use_tc_tiling_on_sc=False
    ),
)
def _emb_lookup(idx_hbm, table_hbm, out_hbm, idx_v, rows_v, out_v, *sems):
    sem_g = sems[:NBUF]
    sem_w = sems[NBUF:]
    wid = lax.axis_index("s") * 2 + lax.axis_index("c")
    iota16 = lax.iota(jnp.int32, 16)

    def out_slice(c):
        b0 = wid * BPW + c // CPB
        f0 = (c % CPB) * CG
        return out_hbm.at[b0, pl.ds(f0, CG)]

    pltpu.sync_copy(idx_hbm.at[pl.ds(wid * NCHUNK, NCHUNK)], idx_v)

    for b in range(NBUF):
        pltpu.async_copy(table_hbm.at[idx_v.at[b]], rows_v.at[b], sem_g[b])

    def quad_body(p, _):
        for b in range(NBUF):
            c = p * NBUF + b
            rv = rows_v.at[b]
            ov = out_v.at[b]

            pltpu.make_async_copy(table_hbm.at[idx_v.at[c]], rv, sem_g[b]).wait()

            @pl.when(p > 0)
            def _():
                pltpu.make_async_copy(ov, out_slice(c - NBUF), sem_w[b]).wait()

            def row_body(l, _):
                ol = jnp.full((16,), l, dtype=jnp.int32)
                for g in range(CG):
                    r = g * L + l
                    og = jnp.full((16,), g, dtype=jnp.int32)
                    v0 = rv[r, pl.ds(0, 16)]
                    v1 = rv[r, pl.ds(16, 16)]
                    plsc.store_scatter(ov, [og, iota16, ol], v0)
                    plsc.store_scatter(ov, [og, iota16 + 16, ol], v1)
                return 0

            lax.fori_loop(0, L, row_body, 0, unroll=2)

            @pl.when(c + NBUF < NCHUNK)
            def _():
                pltpu.async_copy(table_hbm.at[idx_v.at[c + NBUF]], rv, sem_g[b])

            pltpu.async_copy(ov, out_slice(c), sem_w[b])
        return 0

    lax.fori_loop(0, NCHUNK // NBUF, quad_body, 0)

    for b in range(NBUF):
        c = NCHUNK - NBUF + b
        pltpu.make_async_copy(out_v.at[b], out_slice(c), sem_w[b]).wait()


def kernel(inputs, table):
    idx = inputs.reshape(G // CG, IPC).astype(jnp.int32)
    return _emb_lookup(idx, table)
